# baseline (speedup 1.0000x reference)
import jax
import jax.numpy as jnp
from jax import lax
from jax.experimental import pallas as pl
from jax.experimental.pallas import tpu as pltpu
from jax.experimental.pallas import tpu_sc as plsc

_NUM_ROWS = 16384 * 50
_DIM = 32
_NC, _NS = 2, 16
_NW = _NC * _NS
_PER_W = _NUM_ROWS // _NW
_CHUNK = 1600
_NCHUNK = _PER_W // _CHUNK


def _body(table_hbm, idx_hbm, out_hbm,
          idx_v, rows_a, rows_b, g_sem_a, g_sem_b, s_sem_a, s_sem_b):
    wid = lax.axis_index("s") * _NC + lax.axis_index("c")
    base = wid * _PER_W

    def gather(c, rows, sem):
        return pltpu.make_async_copy(table_hbm.at[idx_v.at[c]], rows, sem)

    def store(c, rows, sem):
        return pltpu.make_async_copy(
            rows, out_hbm.at[pl.ds(base + c * _CHUNK, _CHUNK)], sem)

    pltpu.sync_copy(idx_hbm.at[wid, pl.ds(0, 1)], idx_v.at[pl.ds(0, 1)])
    gather(0, rows_a, g_sem_a).start()
    pltpu.sync_copy(idx_hbm.at[wid, pl.ds(1, _NCHUNK - 1)],
                    idx_v.at[pl.ds(1, _NCHUNK - 1)])

    def pair(g, carry):
        c0 = 2 * g
        c1 = c0 + 1

        @pl.when(g >= 1)
        def _():
            store(c1 - 2, rows_b, s_sem_b).wait()
        gather(c1, rows_b, g_sem_b).start()
        gather(c0, rows_a, g_sem_a).wait()
        store(c0, rows_a, s_sem_a).start()

        @pl.when(g < _NCHUNK // 2 - 1)
        def _():
            store(c0, rows_a, s_sem_a).wait()
            gather(c0 + 2, rows_a, g_sem_a).start()
        gather(c1, rows_b, g_sem_b).wait()
        store(c1, rows_b, s_sem_b).start()
        return carry

    lax.fori_loop(0, _NCHUNK // 2, pair, 0)
    store(_NCHUNK - 2, rows_a, s_sem_a).wait()
    store(_NCHUNK - 1, rows_b, s_sem_b).wait()


_gather_call = pl.kernel(
    _body,
    out_type=jax.ShapeDtypeStruct((_NUM_ROWS, _DIM), jnp.float32),
    mesh=plsc.VectorSubcoreMesh(core_axis_name="c", subcore_axis_name="s"),
    scratch_types=[
        pltpu.VMEM((_NCHUNK, _CHUNK), jnp.int32),
        pltpu.VMEM((_CHUNK, _DIM), jnp.float32),
        pltpu.VMEM((_CHUNK, _DIM), jnp.float32),
        pltpu.SemaphoreType.DMA,
        pltpu.SemaphoreType.DMA,
        pltpu.SemaphoreType.DMA,
        pltpu.SemaphoreType.DMA,
    ],
    compiler_params=pltpu.CompilerParams(use_tc_tiling_on_sc=False),
)


def kernel(indices, table):
    flat_idx = indices.reshape(_NW, _NCHUNK, _CHUNK).astype(jnp.int32)
    out = _gather_call(table, flat_idx)
    return out.reshape(indices.shape + (_DIM,))


# speedup vs baseline: 1.0007x; 1.0007x over previous
import jax
import jax.numpy as jnp
from jax import lax
from jax.experimental import pallas as pl
from jax.experimental.pallas import tpu as pltpu
from jax.experimental.pallas import tpu_sc as plsc

_NUM_ROWS = 16384 * 50
_DIM = 32
_NC, _NS = 2, 16
_NW = _NC * _NS
_PER_W = _NUM_ROWS // _NW
_CHUNK = 1280
_NCHUNK = _PER_W // _CHUNK


def _body(table_hbm, idx_hbm, out_hbm,
          idx_v, rows_a, rows_b, g_sem_a, g_sem_b, s_sem_a, s_sem_b):
    wid = lax.axis_index("s") * _NC + lax.axis_index("c")
    base = wid * _PER_W

    def gather(c, rows, sem):
        return pltpu.make_async_copy(table_hbm.at[idx_v.at[c]], rows, sem)

    def store(c, rows, sem):
        return pltpu.make_async_copy(
            rows, out_hbm.at[pl.ds(base + c * _CHUNK, _CHUNK)], sem)

    pltpu.sync_copy(idx_hbm.at[wid, pl.ds(0, 1)], idx_v.at[pl.ds(0, 1)])
    gather(0, rows_a, g_sem_a).start()
    pltpu.sync_copy(idx_hbm.at[wid, pl.ds(1, _NCHUNK - 1)],
                    idx_v.at[pl.ds(1, _NCHUNK - 1)])

    def pair(g, carry):
        c0 = 2 * g
        c1 = c0 + 1

        @pl.when(g >= 1)
        def _():
            store(c1 - 2, rows_b, s_sem_b).wait()
        gather(c1, rows_b, g_sem_b).start()
        gather(c0, rows_a, g_sem_a).wait()
        store(c0, rows_a, s_sem_a).start()

        @pl.when(g < _NCHUNK // 2 - 1)
        def _():
            store(c0, rows_a, s_sem_a).wait()
            gather(c0 + 2, rows_a, g_sem_a).start()
        gather(c1, rows_b, g_sem_b).wait()
        store(c1, rows_b, s_sem_b).start()
        return carry

    lax.fori_loop(0, _NCHUNK // 2, pair, 0)
    store(_NCHUNK - 2, rows_a, s_sem_a).wait()
    store(_NCHUNK - 1, rows_b, s_sem_b).wait()


_gather_call = pl.kernel(
    _body,
    out_type=jax.ShapeDtypeStruct((_NUM_ROWS, _DIM), jnp.float32),
    mesh=plsc.VectorSubcoreMesh(core_axis_name="c", subcore_axis_name="s"),
    scratch_types=[
        pltpu.VMEM((_NCHUNK, _CHUNK), jnp.int32),
        pltpu.VMEM((_CHUNK, _DIM), jnp.float32),
        pltpu.VMEM((_CHUNK, _DIM), jnp.float32),
        pltpu.SemaphoreType.DMA,
        pltpu.SemaphoreType.DMA,
        pltpu.SemaphoreType.DMA,
        pltpu.SemaphoreType.DMA,
    ],
    compiler_params=pltpu.CompilerParams(use_tc_tiling_on_sc=False),
)


def kernel(indices, table):
    flat_idx = indices.reshape(_NW, _NCHUNK, _CHUNK).astype(jnp.int32)
    out = _gather_call(table, flat_idx)
    return out.reshape(indices.shape + (_DIM,))
